# baseline (speedup 1.0000x reference)
import functools

import jax
import jax.numpy as jnp
from jax import lax
from jax.experimental import pallas as pl
from jax.experimental.pallas import tpu as pltpu
from jax.experimental.pallas import tpu_sc as plsc

_B, _Q, _C = 64, 100, 41
_R = _B * _Q
_NS = 16
_RPW = _R // _NS
_NG = _RPW // 16
_TPAD = _RPW + 16
_N = 80
_LN2 = 0.6931471805599453

_mesh = plsc.VectorSubcoreMesh(
    core_axis_name="c", subcore_axis_name="s", num_cores=1, num_subcores=_NS
)


@functools.partial(
    pl.kernel,
    out_type=jax.ShapeDtypeStruct((16,), jnp.float32),
    mesh=_mesh,
    compiler_params=pltpu.CompilerParams(
        needs_layout_passes=False, disable_bounds_checks=True
    ),
    scratch_types=[
        pltpu.VMEM((_RPW, _C), jnp.float32),
        pltpu.VMEM((_TPAD,), jnp.float32),
        pltpu.VMEM((_C,), jnp.float32),
        pltpu.VMEM((_N,), jnp.int32),
        pltpu.VMEM((_N,), jnp.int32),
        pltpu.VMEM((_N,), jnp.int32),
        pltpu.VMEM((32,), jnp.float32),
        pltpu.VMEM((_NS * 32,), jnp.float32),
        pltpu.VMEM_SHARED((_NS * 32,), jnp.float32),
        pltpu.SemaphoreType.DMA,
    ],
)
def _sc_loss(x_hbm, b_hbm, q_hbm, t_hbm, ew_hbm, o_hbm,
             x_v, tc_v, ew_v, b_v, q_v, t_v, part_v, red_v, shared, sem):
    wid = lax.axis_index("s")
    lo = wid * _RPW

    cp = pltpu.async_copy(x_hbm.at[pl.ds(lo, _RPW)], x_v, sem)
    pltpu.sync_copy(b_hbm, b_v)
    pltpu.sync_copy(q_hbm, q_v)
    pltpu.sync_copy(t_hbm, t_v)
    pltpu.sync_copy(ew_hbm, ew_v)

    lanes = lax.broadcasted_iota(jnp.int32, (16,), 0)

    fill = jnp.full((16,), float(_C - 1), jnp.float32)
    for g in range(_TPAD // 16):
        tc_v[pl.ds(g * 16, 16)] = fill
    for v in range(_N // 16):
        bb = b_v[pl.ds(v * 16, 16)]
        qq = q_v[pl.ds(v * 16, 16)]
        tt = t_v[pl.ds(v * 16, 16)].astype(jnp.float32)
        rloc = bb * _Q + qq - lo
        inrange = (rloc >= 0) & (rloc < _RPW)
        trash = _RPW + (lanes & 15)
        for l in range(16):
            idx = jnp.where(inrange & (lanes == l), rloc, trash)
            plsc.store_scatter(tc_v, [idx], tt)

    cp.wait()

    def _group(g, carry):
        nacc, dacc = carry
        row = g * 16 + lanes

        def _gx(cc):
            return plsc.load_gather(x_v, [row, cc])

        cols = [_gx(jnp.full((16,), c, jnp.int32)) for c in range(_C)]
        t = list(cols)
        while len(t) > 1:
            t = [jnp.maximum(t[i], t[i + 1]) for i in range(0, len(t) - 1, 2)] + (
                [t[-1]] if len(t) % 2 else []
            )
        m = t[0]
        t = [jnp.exp(c - m) for c in cols]
        while len(t) > 1:
            t = [t[i] + t[i + 1] for i in range(0, len(t) - 1, 2)] + (
                [t[-1]] if len(t) % 2 else []
            )
        s = t[0]
        bits = plsc.bitcast(s, jnp.int32)
        e = (bits >> 23) - 127
        f = plsc.bitcast((bits & 0x007FFFFF) | 0x3F800000, jnp.float32)
        z = (f - 1.0) / (f + 1.0)
        z2 = z * z
        p = z * (2.0 + z2 * (2.0 / 3.0 + z2 * (2.0 / 5.0 + z2 * (2.0 / 7.0 + z2 * (2.0 / 9.0)))))
        lse = m + e.astype(jnp.float32) * _LN2 + p
        tc = plsc.load_gather(tc_v, [row]).astype(jnp.int32)
        w = plsc.load_gather(ew_v, [tc])
        xtc = _gx(tc)
        nacc = nacc + w * (lse - xtc)
        dacc = dacc + w
        return nacc, dacc

    z16 = jnp.zeros((16,), jnp.float32)
    nacc, dacc = lax.fori_loop(0, _NG, _group, (z16, z16))
    part_v[pl.ds(0, 16)] = nacc
    part_v[pl.ds(16, 16)] = dacc
    pltpu.sync_copy(part_v, shared.at[pl.ds(wid * 32, 32)])
    plsc.subcore_barrier()

    @pl.when(wid == 0)
    def _finish():
        pltpu.sync_copy(shared, red_v)
        ns = jnp.zeros((16,), jnp.float32)
        ds = jnp.zeros((16,), jnp.float32)
        for i in range(_NS):
            ns = ns + red_v[pl.ds(i * 32, 16)]
            ds = ds + red_v[pl.ds(i * 32 + 16, 16)]
        for step in (1, 2, 4, 8):
            part_v[pl.ds(0, 16)] = ns
            part_v[pl.ds(16, 16)] = ds
            ns = ns + plsc.load_gather(part_v, [lanes ^ step])
            ds = ds + plsc.load_gather(part_v, [(lanes ^ step) + 16])
        part_v[pl.ds(0, 16)] = ns / ds
        pltpu.sync_copy(part_v.at[pl.ds(0, 16)], o_hbm)


def kernel(pred_logits, targets, indices_b, indices_q, empty_weight):
    x = pred_logits.reshape(_R, _C)
    out = _sc_loss(x, indices_b, indices_q, targets, empty_weight)
    return out[0]


# speedup vs baseline: 1.2825x; 1.2825x over previous
import functools

import jax
import jax.numpy as jnp
from jax import lax
from jax.experimental import pallas as pl
from jax.experimental.pallas import tpu as pltpu
from jax.experimental.pallas import tpu_sc as plsc

_B, _Q, _C = 64, 100, 41
_R = _B * _Q
_NS = 16
_RPW = _R // _NS
_NG = _RPW // 16
_TPAD = _RPW + 16
_N = 80
_LN2 = 0.6931471805599453

_mesh = plsc.VectorSubcoreMesh(
    core_axis_name="c", subcore_axis_name="s", num_cores=1, num_subcores=_NS
)


@functools.partial(
    pl.kernel,
    out_type=jax.ShapeDtypeStruct((16,), jnp.float32),
    mesh=_mesh,
    compiler_params=pltpu.CompilerParams(
        needs_layout_passes=False, disable_bounds_checks=True
    ),
    scratch_types=[
        pltpu.VMEM((_RPW * _C,), jnp.float32),
        pltpu.VMEM((_TPAD,), jnp.float32),
        pltpu.VMEM((_C,), jnp.float32),
        pltpu.VMEM((_N,), jnp.int32),
        pltpu.VMEM((_N,), jnp.int32),
        pltpu.VMEM((_N,), jnp.int32),
        pltpu.VMEM((32,), jnp.float32),
        pltpu.VMEM((_NS * 32,), jnp.float32),
        pltpu.VMEM_SHARED((_NS * 32,), jnp.float32),
        pltpu.SemaphoreType.DMA,
        pltpu.SemaphoreType.DMA,
    ],
)
def _sc_loss(x_hbm, b_hbm, q_hbm, t_hbm, ew_hbm, o_hbm,
             x_v, tc_v, ew_v, b_v, q_v, t_v, part_v, red_v, shared, sem, sem2):
    wid = lax.axis_index("s")
    lo = wid * _RPW

    cp = pltpu.async_copy(x_hbm.at[pl.ds(lo * _C, _RPW * _C)], x_v, sem)
    cb = pltpu.async_copy(b_hbm, b_v, sem2)
    cq = pltpu.async_copy(q_hbm, q_v, sem2)
    ct = pltpu.async_copy(t_hbm, t_v, sem2)
    cw = pltpu.async_copy(ew_hbm, ew_v, sem2)
    cb.wait()
    cq.wait()
    ct.wait()
    cw.wait()

    lanes = lax.broadcasted_iota(jnp.int32, (16,), 0)

    fill = jnp.full((16,), float(_C - 1), jnp.float32)
    for g in range(_TPAD // 16):
        tc_v[pl.ds(g * 16, 16)] = fill
    for v in range(_N // 16):
        bb = b_v[pl.ds(v * 16, 16)]
        qq = q_v[pl.ds(v * 16, 16)]
        tt = t_v[pl.ds(v * 16, 16)].astype(jnp.float32)
        rloc = bb * _Q + qq - lo
        inrange = (rloc >= 0) & (rloc < _RPW)
        trash = _RPW + (lanes & 15)
        for l in range(16):
            idx = jnp.where(inrange & (lanes == l), rloc, trash)
            plsc.store_scatter(tc_v, [idx], tt)

    cp.wait()

    def _gx(flat):
        return plsc.load_gather(x_v, [flat])

    def _group(g, carry):
        nacc, dacc = carry
        row = g * 16 + lanes
        xb = row * _C
        cols = [_gx(xb + c) for c in range(_C)]
        t = list(cols)
        while len(t) > 1:
            t = [jnp.maximum(t[i], t[i + 1]) for i in range(0, len(t) - 1, 2)] + (
                [t[-1]] if len(t) % 2 else []
            )
        m = t[0]
        t = [jnp.exp(c - m) for c in cols]
        while len(t) > 1:
            t = [t[i] + t[i + 1] for i in range(0, len(t) - 1, 2)] + (
                [t[-1]] if len(t) % 2 else []
            )
        s = t[0]
        bits = plsc.bitcast(s, jnp.int32)
        e = (bits >> 23) - 127
        f = plsc.bitcast((bits & 0x007FFFFF) | 0x3F800000, jnp.float32)
        z = (f - 1.0) / (f + 1.0)
        z2 = z * z
        p = z * (2.0 + z2 * (2.0 / 3.0 + z2 * (2.0 / 5.0 + z2 * (2.0 / 7.0 + z2 * (2.0 / 9.0)))))
        lse = m + e.astype(jnp.float32) * _LN2 + p
        tc = plsc.load_gather(tc_v, [row]).astype(jnp.int32)
        w = plsc.load_gather(ew_v, [tc])
        xtc = _gx(xb + tc)
        nacc = nacc + w * (lse - xtc)
        dacc = dacc + w
        return nacc, dacc

    z16 = jnp.zeros((16,), jnp.float32)
    nacc, dacc = lax.fori_loop(0, _NG, _group, (z16, z16))
    part_v[pl.ds(0, 16)] = nacc
    part_v[pl.ds(16, 16)] = dacc
    pltpu.sync_copy(part_v, shared.at[pl.ds(wid * 32, 32)])
    plsc.subcore_barrier()

    @pl.when(wid == 0)
    def _finish():
        pltpu.sync_copy(shared, red_v)
        ns = jnp.zeros((16,), jnp.float32)
        ds = jnp.zeros((16,), jnp.float32)
        for i in range(_NS):
            ns = ns + red_v[pl.ds(i * 32, 16)]
            ds = ds + red_v[pl.ds(i * 32 + 16, 16)]
        for step in (1, 2, 4, 8):
            part_v[pl.ds(0, 16)] = ns
            part_v[pl.ds(16, 16)] = ds
            ns = ns + plsc.load_gather(part_v, [lanes ^ step])
            ds = ds + plsc.load_gather(part_v, [(lanes ^ step) + 16])
        part_v[pl.ds(0, 16)] = ns / ds
        pltpu.sync_copy(part_v.at[pl.ds(0, 16)], o_hbm)


def kernel(pred_logits, targets, indices_b, indices_q, empty_weight):
    x = pred_logits.reshape(-1)
    out = _sc_loss(x, indices_b, indices_q, targets, empty_weight)
    return out[0]
